# baseline (speedup 1.0000x reference)
import functools

import jax
import jax.numpy as jnp
from jax import lax
from jax.experimental import pallas as pl
from jax.experimental.pallas import tpu as pltpu
from jax.experimental.pallas import tpu_sc as plsc

B = 16384
D = 32
H = 64
NC = 2
NS = 16
NW = NC * NS
ROWS = B // NW


def _sc_gather_body(uid_hbm, pid_hbm, iid_hbm, ut_hbm, pt_hbm, it_hbm,
                    u_out, p_out, i_out, idx_v, rows_v, sem):
    wid = lax.axis_index("s") * NC + lax.axis_index("c")
    base = wid * ROWS
    for ids_hbm, tab_hbm, out_hbm in ((uid_hbm, ut_hbm, u_out),
                                      (pid_hbm, pt_hbm, p_out),
                                      (iid_hbm, it_hbm, i_out)):
        pltpu.sync_copy(ids_hbm.at[pl.ds(base, ROWS)], idx_v)
        pltpu.async_copy(tab_hbm.at[idx_v], rows_v, sem).wait()
        pltpu.sync_copy(rows_v, out_hbm.at[pl.ds(base, ROWS)])


_sc_gather = pl.kernel(
    _sc_gather_body,
    out_type=(
        jax.ShapeDtypeStruct((B, D), jnp.float32),
        jax.ShapeDtypeStruct((B, D), jnp.float32),
        jax.ShapeDtypeStruct((B, D), jnp.float32),
    ),
    mesh=plsc.VectorSubcoreMesh(core_axis_name="c", subcore_axis_name="s"),
    scratch_types=[
        pltpu.VMEM((ROWS,), jnp.int32),
        pltpu.VMEM((ROWS, D), jnp.float32),
        pltpu.SemaphoreType.DMA,
    ],
    compiler_params=pltpu.CompilerParams(use_tc_tiling_on_sc=False),
)


def _mlp_body(u_ref, p_ref, i_ref, w1u_ref, w1p_ref, w1i_ref, b1_ref,
              w2_ref, b2_ref, out_ref):
    h = (jnp.dot(u_ref[...], w1u_ref[...], preferred_element_type=jnp.float32)
         + jnp.dot(p_ref[...], w1p_ref[...], preferred_element_type=jnp.float32)
         + jnp.dot(i_ref[...], w1i_ref[...], preferred_element_type=jnp.float32)
         + b1_ref[...])
    h = jnp.maximum(h, 0.0)
    out_ref[...] = (jnp.dot(h, w2_ref[...], preferred_element_type=jnp.float32)
                    + b2_ref[...])


BS = 2048


def _tc_mlp(u, p, i, w1u, w1p, w1i, b1, w2, b2):
    grid = (B // BS,)
    emb_spec = pl.BlockSpec((BS, D), lambda j: (j, 0))
    full = lambda shape: pl.BlockSpec(shape, lambda j: (0, 0))
    return pl.pallas_call(
        _mlp_body,
        grid=grid,
        in_specs=[emb_spec, emb_spec, emb_spec,
                  full((D, H)), full((D, H)), full((D, H)), full((1, H)),
                  full((H, 1)), full((1, 1))],
        out_specs=pl.BlockSpec((BS, 1), lambda j: (j, 0)),
        out_shape=jax.ShapeDtypeStruct((B, 1), jnp.float32),
    )(u, p, i, w1u, w1p, w1i, b1, w2, b2)


def kernel(user_ids, product_ids, interaction_ids, user_table, product_table,
           interaction_table, W1, b1, W2, b2):
    uid = user_ids.astype(jnp.int32)
    pid = product_ids.astype(jnp.int32)
    iid = interaction_ids.astype(jnp.int32)
    u, p, i = _sc_gather(uid, pid, iid, user_table, product_table,
                         interaction_table)
    w1u, w1p, w1i = W1[:D], W1[D:2 * D], W1[2 * D:]
    return _tc_mlp(u, p, i, w1u, w1p, w1i, b1.reshape(1, H), W2,
                   b2.reshape(1, 1))
